# baseline (speedup 1.0000x reference)
import jax
import jax.numpy as jnp
from jax.experimental import pallas as pl
from jax.experimental.pallas import tpu as pltpu

_BT = 512


def _stats_kernel(seq_ref, x_ref, out_ref, *, bt, blocks_per_seq):
    i = pl.program_id(0)

    @pl.when(i == 0)
    def _init():
        out_ref[...] = jnp.zeros_like(out_ref)

    b = i // blocks_per_seq
    start = (i % blocks_per_seq) * bt
    seqlen = seq_ref[b]

    @pl.when(seqlen > start)
    def _acc():
        pos = start + jax.lax.broadcasted_iota(jnp.int32, (bt, 1), 0)
        m = (pos < seqlen).astype(jnp.float32)
        x = x_ref[...]
        xm = x * m
        out_ref[0:1, :] += jnp.sum(xm, axis=0, keepdims=True)
        out_ref[1:2, :] += jnp.sum(xm * x, axis=0, keepdims=True)
        out_ref[2:3, :] += jnp.sum(m)


def _apply_kernel(seq_ref, x_ref, stats_ref, gb_ref, w_ref, bb_ref, out_ref,
                  *, bt, blocks_per_seq, out_dim):
    i = pl.program_id(0)
    b = i // blocks_per_seq
    start = (i % blocks_per_seq) * bt
    seqlen = seq_ref[b]
    bvec = bb_ref[0:1, :]

    @pl.when(seqlen > start)
    def _valid():
        cnt = jnp.maximum(jnp.max(stats_ref[2:3, :]), 1.0)
        mean = stats_ref[0:1, :] / cnt
        var = stats_ref[1:2, :] / cnt - mean * mean
        scale = jax.lax.rsqrt(var + 1e-5) * gb_ref[0:1, :]
        shift = gb_ref[1:2, :] - mean * scale
        xn = x_ref[...] * scale + shift
        y = jax.lax.dot_general(xn, w_ref[...], (((1,), (1,)), ((), ())),
                                preferred_element_type=jnp.float32)
        y = y + bvec
        pos = start + jax.lax.broadcasted_iota(jnp.int32, (bt, 1), 0)
        y = jnp.where(pos < seqlen, y, bvec)
        y = y * jax.lax.rsqrt(jnp.sum(y * y, axis=1, keepdims=True) + 1e-12)
        out_ref[...] = y

    @pl.when(seqlen <= start)
    def _pad():
        bhat = bvec * jax.lax.rsqrt(jnp.sum(bvec * bvec) + 1e-12)
        out_ref[...] = jnp.broadcast_to(bhat, (bt, out_dim))


def kernel(payload, seq_lens, gamma, beta, W, b):
    B, T, D = payload.shape
    O = W.shape[0]
    bt = _BT
    blocks_per_seq = T // bt
    nb = (B * T) // bt

    x2d = payload.reshape(B * T, D)
    seq = seq_lens.astype(jnp.int32)

    import functools
    stats = pl.pallas_call(
        functools.partial(_stats_kernel, bt=bt, blocks_per_seq=blocks_per_seq),
        grid=(nb,),
        in_specs=[
            pl.BlockSpec(memory_space=pltpu.SMEM),
            pl.BlockSpec((bt, D), lambda i: (i, 0)),
        ],
        out_specs=pl.BlockSpec((8, D), lambda i: (0, 0)),
        out_shape=jax.ShapeDtypeStruct((8, D), jnp.float32),
    )(seq, x2d)

    gb = jnp.zeros((8, D), jnp.float32).at[0].set(gamma).at[1].set(beta)
    bb = jnp.broadcast_to(b, (8, O))

    y = pl.pallas_call(
        functools.partial(_apply_kernel, bt=bt, blocks_per_seq=blocks_per_seq,
                          out_dim=O),
        grid=(nb,),
        in_specs=[
            pl.BlockSpec(memory_space=pltpu.SMEM),
            pl.BlockSpec((bt, D), lambda i: (i, 0)),
            pl.BlockSpec((8, D), lambda i: (0, 0)),
            pl.BlockSpec((8, D), lambda i: (0, 0)),
            pl.BlockSpec((O, D), lambda i: (0, 0)),
            pl.BlockSpec((8, O), lambda i: (0, 0)),
        ],
        out_specs=pl.BlockSpec((bt, O), lambda i: (i, 0)),
        out_shape=jax.ShapeDtypeStruct((B * T, O), jnp.float32),
    )(seq, x2d, stats, gb, W, bb)

    return y.reshape(B, T, O)


# speedup vs baseline: 1.6517x; 1.6517x over previous
import functools

import jax
import jax.numpy as jnp
from jax.experimental import pallas as pl
from jax.experimental.pallas import tpu as pltpu

_BT = 1024


def _stats_kernel(eff_ref, seq_ref, x_ref, gb_ref, w_ref, bp_ref,
                  w2_ref, aux_ref, acc_ref, *, bt, bpb, nb):
    i = pl.program_id(0)

    @pl.when(i == 0)
    def _init():
        acc_ref[...] = jnp.zeros_like(acc_ref)

    b = i // bpb
    start = (i % bpb) * bt
    seqlen = seq_ref[b]
    full = seqlen >= start + bt

    @pl.when(full)
    def _full():
        x = x_ref[...]
        acc_ref[0:1, :] += jnp.sum(x, axis=0, keepdims=True)
        acc_ref[1:2, :] += jnp.sum(x * x, axis=0, keepdims=True)
        acc_ref[2:3, :] += float(bt)

    @pl.when(jnp.logical_and(seqlen > start, jnp.logical_not(full)))
    def _partial():
        pos = start + jax.lax.broadcasted_iota(jnp.int32, (bt, 1), 0)
        m = (pos < seqlen).astype(jnp.float32)
        x = x_ref[...]
        xm = x * m
        acc_ref[0:1, :] += jnp.sum(xm, axis=0, keepdims=True)
        acc_ref[1:2, :] += jnp.sum(xm * x, axis=0, keepdims=True)
        acc_ref[2:3, :] += jnp.sum(m)

    @pl.when(i == nb - 1)
    def _finalize():
        cnt = jnp.maximum(jnp.max(acc_ref[2:3, :]), 1.0)
        mean = acc_ref[0:1, :] / cnt
        var = acc_ref[1:2, :] / cnt - mean * mean
        scale = jax.lax.rsqrt(var + 1e-5) * gb_ref[0:1, :]
        shift = gb_ref[1:2, :] - mean * scale
        w2_ref[...] = w_ref[...] * scale
        brow = bp_ref[0:1, :]
        b2 = brow + jax.lax.dot_general(
            shift, w_ref[...], (((1,), (1,)), ((), ())),
            preferred_element_type=jnp.float32)
        bhat = brow * jax.lax.rsqrt(jnp.sum(brow * brow) + 1e-12)
        o = brow.shape[1]
        aux_ref[...] = jnp.concatenate(
            [b2, bhat, jnp.zeros((6, o), jnp.float32)], axis=0)


def _apply_kernel(eff_ref, seq_ref, x_ref, w2_ref, aux_ref, out_ref,
                  *, bt, bpb, out_dim):
    i = pl.program_id(0)
    b = i // bpb
    start = (i % bpb) * bt
    seqlen = seq_ref[b]
    full = seqlen >= start + bt

    @pl.when(full)
    def _full():
        y = jax.lax.dot_general(
            x_ref[...], w2_ref[...], (((1,), (1,)), ((), ())),
            preferred_element_type=jnp.float32) + aux_ref[0:1, :]
        out_ref[...] = y * jax.lax.rsqrt(
            jnp.sum(y * y, axis=1, keepdims=True) + 1e-12)

    @pl.when(jnp.logical_and(seqlen > start, jnp.logical_not(full)))
    def _partial():
        y = jax.lax.dot_general(
            x_ref[...], w2_ref[...], (((1,), (1,)), ((), ())),
            preferred_element_type=jnp.float32) + aux_ref[0:1, :]
        y = y * jax.lax.rsqrt(jnp.sum(y * y, axis=1, keepdims=True) + 1e-12)
        pos = start + jax.lax.broadcasted_iota(jnp.int32, (bt, 1), 0)
        out_ref[...] = jnp.where(pos < seqlen, y, aux_ref[1:2, :])

    @pl.when(seqlen <= start)
    def _pad():
        out_ref[...] = jnp.broadcast_to(aux_ref[1:2, :], (bt, out_dim))


def kernel(payload, seq_lens, gamma, beta, W, b):
    B, T, D = payload.shape
    O = W.shape[0]
    bt = _BT
    bpb = T // bt
    nb = (B * T) // bt

    x2d = payload.reshape(B * T, D)
    seq = seq_lens.astype(jnp.int32)

    blk = jnp.arange(nb, dtype=jnp.int32)
    starts = (blk % bpb) * bt
    valid = seq[blk // bpb] > starts
    eff = jnp.maximum(jax.lax.cummax(jnp.where(valid, blk, -1)), 0)
    eff = eff.astype(jnp.int32)

    gb = jnp.zeros((8, D), jnp.float32).at[0].set(gamma).at[1].set(beta)
    bp = jnp.broadcast_to(b, (8, O))

    w2, aux = pl.pallas_call(
        functools.partial(_stats_kernel, bt=bt, bpb=bpb, nb=nb),
        grid_spec=pltpu.PrefetchScalarGridSpec(
            num_scalar_prefetch=2,
            grid=(nb,),
            in_specs=[
                pl.BlockSpec((bt, D), lambda i, eff, s: (eff[i], 0)),
                pl.BlockSpec((8, D), lambda i, eff, s: (0, 0)),
                pl.BlockSpec((O, D), lambda i, eff, s: (0, 0)),
                pl.BlockSpec((8, O), lambda i, eff, s: (0, 0)),
            ],
            out_specs=[
                pl.BlockSpec((O, D), lambda i, eff, s: (0, 0)),
                pl.BlockSpec((8, O), lambda i, eff, s: (0, 0)),
            ],
            scratch_shapes=[pltpu.VMEM((8, D), jnp.float32)],
        ),
        out_shape=[
            jax.ShapeDtypeStruct((O, D), jnp.float32),
            jax.ShapeDtypeStruct((8, O), jnp.float32),
        ],
    )(eff, seq, x2d, gb, W, bp)

    y = pl.pallas_call(
        functools.partial(_apply_kernel, bt=bt, bpb=bpb, out_dim=O),
        grid_spec=pltpu.PrefetchScalarGridSpec(
            num_scalar_prefetch=2,
            grid=(nb,),
            in_specs=[
                pl.BlockSpec((bt, D), lambda i, eff, s: (eff[i], 0)),
                pl.BlockSpec((O, D), lambda i, eff, s: (0, 0)),
                pl.BlockSpec((8, O), lambda i, eff, s: (0, 0)),
            ],
            out_specs=pl.BlockSpec((bt, O), lambda i, eff, s: (i, 0)),
        ),
        out_shape=jax.ShapeDtypeStruct((B * T, O), jnp.float32),
    )(eff, seq, x2d, w2, aux)

    return y.reshape(B, T, O)


# speedup vs baseline: 1.7295x; 1.0471x over previous
import functools

import jax
import jax.numpy as jnp
from jax.experimental import pallas as pl
from jax.experimental.pallas import tpu as pltpu

_BT = 1024


def _fused_kernel(eff_ref, seq_ref, x_ref, g_ref, bet_ref, w_ref, b_ref,
                  out_ref, acc_ref, w2_ref, aux_ref, *, bt, bpb, nb, out_dim):
    i = pl.program_id(0)
    phase1 = i < nb
    j = jnp.where(phase1, i, i - nb)
    b = j // bpb
    start = (j % bpb) * bt
    seqlen = seq_ref[b]
    valid = seqlen > start
    full = seqlen >= start + bt

    @pl.when(i == 0)
    def _init():
        acc_ref[...] = jnp.zeros_like(acc_ref)

    @pl.when(jnp.logical_and(phase1, valid))
    def _stats():
        pos = start + jax.lax.broadcasted_iota(jnp.int32, (1, bt), 1)
        m = (pos < seqlen).astype(jnp.float32)
        x = x_ref[...]
        acc_ref[0:1, :] += jax.lax.dot_general(
            m, x, (((1,), (0,)), ((), ())),
            preferred_element_type=jnp.float32)
        acc_ref[1:2, :] += jax.lax.dot_general(
            m, x * x, (((1,), (0,)), ((), ())),
            preferred_element_type=jnp.float32)
        acc_ref[2:3, :] += jnp.sum(m)

    @pl.when(i == nb - 1)
    def _finalize():
        cnt = jnp.maximum(jnp.max(acc_ref[2:3, :]), 1.0)
        mean = acc_ref[0:1, :] / cnt
        var = acc_ref[1:2, :] / cnt - mean * mean
        scale = jax.lax.rsqrt(var + 1e-5) * g_ref[...][None, :]
        shift = bet_ref[...][None, :] - mean * scale
        w2_ref[...] = w_ref[...] * scale
        brow = b_ref[...][None, :]
        b2 = brow + jax.lax.dot_general(
            shift, w_ref[...], (((1,), (1,)), ((), ())),
            preferred_element_type=jnp.float32)
        bhat = brow * jax.lax.rsqrt(jnp.sum(brow * brow) + 1e-12)
        aux_ref[...] = jnp.concatenate(
            [b2, bhat, jnp.zeros((6, out_dim), jnp.float32)], axis=0)

    phase2 = jnp.logical_not(phase1)

    @pl.when(jnp.logical_and(phase2, full))
    def _apply_full():
        y = jax.lax.dot_general(
            x_ref[...], w2_ref[...], (((1,), (1,)), ((), ())),
            preferred_element_type=jnp.float32) + aux_ref[0:1, :]
        out_ref[...] = y * jax.lax.rsqrt(
            jnp.sum(y * y, axis=1, keepdims=True) + 1e-12)

    @pl.when(jnp.logical_and(phase2, jnp.logical_and(valid, jnp.logical_not(full))))
    def _apply_partial():
        y = jax.lax.dot_general(
            x_ref[...], w2_ref[...], (((1,), (1,)), ((), ())),
            preferred_element_type=jnp.float32) + aux_ref[0:1, :]
        y = y * jax.lax.rsqrt(jnp.sum(y * y, axis=1, keepdims=True) + 1e-12)
        pos = start + jax.lax.broadcasted_iota(jnp.int32, (bt, 1), 0)
        out_ref[...] = jnp.where(pos < seqlen, y, aux_ref[1:2, :])

    @pl.when(jnp.logical_and(phase2, jnp.logical_not(valid)))
    def _apply_pad():
        out_ref[...] = jnp.broadcast_to(aux_ref[1:2, :], (bt, out_dim))


def kernel(payload, seq_lens, gamma, beta, W, b):
    B, T, D = payload.shape
    O = W.shape[0]
    bt = _BT
    bpb = T // bt
    nb = (B * T) // bt

    x2d = payload.reshape(B * T, D)
    seq = seq_lens.astype(jnp.int32)

    blk = jnp.arange(nb, dtype=jnp.int32)
    starts = (blk % bpb) * bt
    valid = seq[blk // bpb] > starts
    eff = jnp.maximum(jax.lax.cummax(jnp.where(valid, blk, -1)), 0)
    eff = eff.astype(jnp.int32)

    def _xmap(i, eff, s):
        return (eff[jnp.where(i < nb, i, i - nb)], 0)

    def _omap(i, eff, s):
        return (jnp.where(i < nb, 0, i - nb), 0)

    y = pl.pallas_call(
        functools.partial(_fused_kernel, bt=bt, bpb=bpb, nb=nb, out_dim=O),
        grid_spec=pltpu.PrefetchScalarGridSpec(
            num_scalar_prefetch=2,
            grid=(2 * nb,),
            in_specs=[
                pl.BlockSpec((bt, D), _xmap),
                pl.BlockSpec((D,), lambda i, eff, s: (0,)),
                pl.BlockSpec((D,), lambda i, eff, s: (0,)),
                pl.BlockSpec((O, D), lambda i, eff, s: (0, 0)),
                pl.BlockSpec((O,), lambda i, eff, s: (0,)),
            ],
            out_specs=pl.BlockSpec((bt, O), _omap),
            scratch_shapes=[
                pltpu.VMEM((8, D), jnp.float32),
                pltpu.VMEM((O, D), jnp.float32),
                pltpu.VMEM((8, O), jnp.float32),
            ],
        ),
        out_shape=jax.ShapeDtypeStruct((B * T, O), jnp.float32),
        compiler_params=pltpu.CompilerParams(
            dimension_semantics=("arbitrary",)),
    )(eff, seq, x2d, gamma, beta, W, b)

    return y.reshape(B, T, O)


# speedup vs baseline: 1.9166x; 1.1082x over previous
import functools

import jax
import jax.numpy as jnp
from jax.experimental import pallas as pl
from jax.experimental.pallas import tpu as pltpu

_BT = 2048


def _fused_kernel(eff_ref, seq_ref, x_ref, g_ref, bet_ref, w_ref, b_ref,
                  out_ref, acc_ref, w2_ref, aux_ref, *, bt, bpb, nb, out_dim):
    i = pl.program_id(0)
    phase1 = i < nb
    j = jnp.where(phase1, i, i - nb)
    b = j // bpb
    start = (j % bpb) * bt
    seqlen = seq_ref[b]
    valid = seqlen > start
    full = seqlen >= start + bt

    @pl.when(i == 0)
    def _init():
        acc_ref[...] = jnp.zeros_like(acc_ref)

    @pl.when(jnp.logical_and(phase1, valid))
    def _stats():
        pos = start + jax.lax.broadcasted_iota(jnp.int32, (1, bt), 1)
        m = (pos < seqlen).astype(jnp.float32)
        x = x_ref[...]
        acc_ref[0:1, :] += jax.lax.dot_general(
            m, x, (((1,), (0,)), ((), ())),
            preferred_element_type=jnp.float32)
        acc_ref[1:2, :] += jax.lax.dot_general(
            m, x * x, (((1,), (0,)), ((), ())),
            preferred_element_type=jnp.float32)
        acc_ref[2:3, :] += jnp.sum(m)

    @pl.when(i == nb - 1)
    def _finalize():
        cnt = jnp.maximum(jnp.max(acc_ref[2:3, :]), 1.0)
        mean = acc_ref[0:1, :] / cnt
        var = acc_ref[1:2, :] / cnt - mean * mean
        scale = jax.lax.rsqrt(var + 1e-5) * g_ref[...][None, :]
        shift = bet_ref[...][None, :] - mean * scale
        w2_ref[...] = w_ref[...] * scale
        brow = b_ref[...][None, :]
        b2 = brow + jax.lax.dot_general(
            shift, w_ref[...], (((1,), (1,)), ((), ())),
            preferred_element_type=jnp.float32)
        bhat = brow * jax.lax.rsqrt(jnp.sum(brow * brow) + 1e-12)
        aux_ref[...] = jnp.concatenate(
            [b2, bhat, jnp.zeros((6, out_dim), jnp.float32)], axis=0)

    phase2 = jnp.logical_not(phase1)

    @pl.when(jnp.logical_and(phase2, full))
    def _apply_full():
        y = jax.lax.dot_general(
            x_ref[...], w2_ref[...], (((1,), (1,)), ((), ())),
            preferred_element_type=jnp.float32) + aux_ref[0:1, :]
        out_ref[...] = y * jax.lax.rsqrt(
            jnp.sum(y * y, axis=1, keepdims=True) + 1e-12)

    @pl.when(jnp.logical_and(phase2, jnp.logical_and(valid, jnp.logical_not(full))))
    def _apply_partial():
        y = jax.lax.dot_general(
            x_ref[...], w2_ref[...], (((1,), (1,)), ((), ())),
            preferred_element_type=jnp.float32) + aux_ref[0:1, :]
        y = y * jax.lax.rsqrt(jnp.sum(y * y, axis=1, keepdims=True) + 1e-12)
        pos = start + jax.lax.broadcasted_iota(jnp.int32, (bt, 1), 0)
        out_ref[...] = jnp.where(pos < seqlen, y, aux_ref[1:2, :])

    @pl.when(jnp.logical_and(phase2, jnp.logical_not(valid)))
    def _apply_pad():
        out_ref[...] = jnp.broadcast_to(aux_ref[1:2, :], (bt, out_dim))


def kernel(payload, seq_lens, gamma, beta, W, b):
    B, T, D = payload.shape
    O = W.shape[0]
    bt = _BT
    bpb = T // bt
    nb = (B * T) // bt

    x2d = payload.reshape(B * T, D)
    seq = seq_lens.astype(jnp.int32)

    blk = jnp.arange(nb, dtype=jnp.int32)
    starts = (blk % bpb) * bt
    valid = seq[blk // bpb] > starts
    eff = jnp.maximum(jax.lax.cummax(jnp.where(valid, blk, -1)), 0)
    eff = eff.astype(jnp.int32)

    def _xmap(i, eff, s):
        return (eff[jnp.where(i < nb, i, i - nb)], 0)

    def _omap(i, eff, s):
        return (jnp.where(i < nb, 0, i - nb), 0)

    y = pl.pallas_call(
        functools.partial(_fused_kernel, bt=bt, bpb=bpb, nb=nb, out_dim=O),
        grid_spec=pltpu.PrefetchScalarGridSpec(
            num_scalar_prefetch=2,
            grid=(2 * nb,),
            in_specs=[
                pl.BlockSpec((bt, D), _xmap),
                pl.BlockSpec((D,), lambda i, eff, s: (0,)),
                pl.BlockSpec((D,), lambda i, eff, s: (0,)),
                pl.BlockSpec((O, D), lambda i, eff, s: (0, 0)),
                pl.BlockSpec((O,), lambda i, eff, s: (0,)),
            ],
            out_specs=pl.BlockSpec((bt, O), _omap),
            scratch_shapes=[
                pltpu.VMEM((8, D), jnp.float32),
                pltpu.VMEM((O, D), jnp.float32),
                pltpu.VMEM((8, O), jnp.float32),
            ],
        ),
        out_shape=jax.ShapeDtypeStruct((B * T, O), jnp.float32),
        compiler_params=pltpu.CompilerParams(
            dimension_semantics=("arbitrary",)),
    )(eff, seq, x2d, gamma, beta, W, b)

    return y.reshape(B, T, O)
